# baseline (speedup 1.0000x reference)
import jax
import jax.numpy as jnp
from jax.experimental import pallas as pl
from jax.experimental.pallas import tpu as pltpu


def _bce_kernel(label_ref, logits_ref, mask_ref, out_ref, acc_ref):
    i = pl.program_id(0)

    @pl.when(i == 0)
    def _init():
        acc_ref[0] = 0.0
        acc_ref[1] = 0.0

    y = label_ref[...]
    p = logits_ref[...]
    m = (mask_ref[...] == 1).astype(jnp.float32)
    log_p = jnp.maximum(jnp.log(p), -100.0)
    log_1mp = jnp.maximum(jnp.log(1.0 - p), -100.0)
    bce = y * log_p + (1.0 - y) * log_1mp
    acc_ref[0] += jnp.sum(bce * m)
    acc_ref[1] += jnp.sum(m)

    @pl.when(i == pl.num_programs(0) - 1)
    def _fin():
        out_ref[0] = -acc_ref[0] / acc_ref[1]


def kernel(label, logits, mask):
    n = label.size
    cols = 1024
    rows = n // cols
    grid = 8
    blk = rows // grid

    label2 = label.reshape(rows, cols)
    logits2 = logits.reshape(rows, cols)
    mask2 = mask.astype(jnp.int32).reshape(rows, cols)

    out = pl.pallas_call(
        _bce_kernel,
        grid=(grid,),
        in_specs=[
            pl.BlockSpec((blk, cols), lambda i: (i, 0)),
            pl.BlockSpec((blk, cols), lambda i: (i, 0)),
            pl.BlockSpec((blk, cols), lambda i: (i, 0)),
        ],
        out_specs=pl.BlockSpec(memory_space=pltpu.SMEM),
        out_shape=jax.ShapeDtypeStruct((1,), jnp.float32),
        scratch_shapes=[pltpu.SMEM((2,), jnp.float32)],
    )(label2, logits2, mask2)
    return out[0]


# speedup vs baseline: 1.6932x; 1.6932x over previous
import jax
import jax.numpy as jnp
from jax.experimental import pallas as pl
from jax.experimental.pallas import tpu as pltpu


def _bce_kernel(label_ref, logits_ref, mask_ref, out_ref, acc_ref):
    i = pl.program_id(0)

    @pl.when(i == 0)
    def _init():
        acc_ref[0] = 0.0
        acc_ref[1] = 0.0

    y = label_ref[...]
    p = logits_ref[...]
    msel = mask_ref[...] == 1
    log_p = jnp.maximum(jnp.log(p), -100.0)
    log_1mp = jnp.maximum(jnp.log(1.0 - p), -100.0)
    bce = y * log_p + (1.0 - y) * log_1mp
    acc_ref[0] += jnp.sum(jnp.where(msel, bce, 0.0))
    acc_ref[1] += jnp.sum(jnp.where(msel, 1.0, 0.0))

    @pl.when(i == pl.num_programs(0) - 1)
    def _fin():
        out_ref[0] = -acc_ref[0] / acc_ref[1]


def kernel(label, logits, mask):
    B, L = label.shape
    grid = 16
    blk = B // grid

    out = pl.pallas_call(
        _bce_kernel,
        grid=(grid,),
        in_specs=[
            pl.BlockSpec((blk, L), lambda i: (i, 0)),
            pl.BlockSpec((blk, L), lambda i: (i, 0)),
            pl.BlockSpec((blk, L), lambda i: (i, 0)),
        ],
        out_specs=pl.BlockSpec(memory_space=pltpu.SMEM),
        out_shape=jax.ShapeDtypeStruct((1,), jnp.float32),
        scratch_shapes=[pltpu.SMEM((2,), jnp.float32)],
    )(label, logits, mask.astype(jnp.int32))
    return out[0]
